# baseline (speedup 1.0000x reference)
import functools

import jax
import jax.numpy as jnp
from jax import lax
from jax.experimental import pallas as pl
from jax.experimental.pallas import tpu as pltpu
from jax.experimental.pallas import tpu_sc as plsc

N = 10000
D = 128
E = 320000
NC = 2
NS = 16
NW = NC * NS
EPW = E // NW
NP = 10240
RPT = NP // NS

_P = lax.Precision.HIGHEST

_mesh = plsc.VectorSubcoreMesh(
    core_axis_name="c", subcore_axis_name="s", num_cores=NC, num_subcores=NS)


def _seg_plan(bsz, cpp_req=None, mode=None):
    del cpp_req, mode
    nch = -(-EPW // bsz)
    return nch, nch * bsz, nch


def _make_seg(bsz, cpp_req=None, mode=None):
    nch, epwp, _ = _seg_plan(bsz)

    scratch = [
        pltpu.VMEM((nch, bsz), jnp.int32),
        pltpu.VMEM((nch, bsz), jnp.int32),
        pltpu.VMEM((bsz, D), jnp.float32),
        pltpu.VMEM_SHARED((NP, D), jnp.float32),
        pltpu.SemaphoreType.DMA,
    ]

    def body_fn(h_hbm, src_hbm, dst_hbm, out_hbm,
                src_v, dst_v, rows_a, acc_sh, sem_a):
        c = lax.axis_index("c")
        s = lax.axis_index("s")
        wid = c * NS + s

        def zr(i, _):
            rows_a[i // 8, pl.ds((i % 8) * 16, 16)] = jnp.zeros(
                (16,), jnp.float32)
            return 0

        lax.fori_loop(0, bsz * (D // 16), zr, 0)
        base = s * RPT
        for j in range(RPT // bsz):
            pltpu.sync_copy(rows_a, acc_sh.at[pl.ds(base + j * bsz, bsz)])
        rem = RPT % bsz
        if rem:
            pltpu.sync_copy(rows_a.at[pl.ds(0, rem)],
                            acc_sh.at[pl.ds(base + (RPT // bsz) * bsz, rem)])
        plsc.subcore_barrier()

        pltpu.sync_copy(src_hbm.at[wid], src_v)
        pltpu.sync_copy(dst_hbm.at[wid], dst_v)

        def sbody(i, _):
            pltpu.async_copy(h_hbm.at[src_v.at[i]], rows_a, sem_a).wait()
            pltpu.sync_copy(rows_a, acc_sh.at[dst_v.at[i]], add=True)
            return 0

        lax.fori_loop(0, nch, sbody, 0)
        plsc.subcore_barrier()
        pltpu.sync_copy(acc_sh.at[pl.ds(base, RPT)],
                        out_hbm.at[c, pl.ds(base, RPT)])

    return pl.kernel(
        body_fn,
        out_type=jax.ShapeDtypeStruct((NC, NP, D), jnp.float32),
        mesh=_mesh,
        scratch_types=scratch,
    )


def _pack_edges(edge_index, bsz, cpp_req=None, mode=None):
    nch, epwp, _ = _seg_plan(bsz)
    srcw = edge_index[0].reshape(NW, EPW)
    dstw = edge_index[1].reshape(NW, EPW)
    if epwp > EPW:
        sp = jnp.zeros((NW, epwp - EPW), jnp.int32)
        dp = jnp.full((NW, epwp - EPW), NP - 1, jnp.int32)
        srcw = jnp.concatenate([srcw, sp], 1)
        dstw = jnp.concatenate([dstw, dp], 1)
    return srcw.reshape(NW, nch, bsz), dstw.reshape(NW, nch, bsz)


_DEG_B = 80
_DEG_NCH = EPW // _DEG_B


_DEG_W = 32


@functools.partial(
    pl.kernel,
    out_type=jax.ShapeDtypeStruct((NC, NP, _DEG_W), jnp.float32),
    mesh=_mesh,
    scratch_types=[
        pltpu.VMEM((_DEG_NCH, _DEG_B), jnp.int32),
        pltpu.VMEM((_DEG_B, _DEG_W), jnp.float32),
        pltpu.VMEM_SHARED((NP, _DEG_W), jnp.float32),
    ],
)
def _deg_sc(dst_hbm, out_hbm, dst_v, ones_v, acc_sh):
    c = lax.axis_index("c")
    s = lax.axis_index("s")
    wid = c * NS + s

    pltpu.sync_copy(dst_hbm.at[wid], dst_v)

    nsl = _DEG_W // 16

    def zr(i, _):
        ones_v[i // nsl, pl.ds((i % nsl) * 16, 16)] = jnp.zeros(
            (16,), jnp.float32)
        return 0

    lax.fori_loop(0, _DEG_B * nsl, zr, 0)

    base = s * RPT
    for j in range(RPT // _DEG_B):
        pltpu.sync_copy(ones_v, acc_sh.at[pl.ds(base + j * _DEG_B, _DEG_B)])

    def fl(i, _):
        ones_v[i // nsl, pl.ds((i % nsl) * 16, 16)] = jnp.full(
            (16,), 1.0, jnp.float32)
        return 0

    lax.fori_loop(0, _DEG_B * nsl, fl, 0)
    plsc.subcore_barrier()

    def body(i, _):
        pltpu.sync_copy(ones_v, acc_sh.at[dst_v.at[i]], add=True)
        return 0

    lax.fori_loop(0, _DEG_NCH, body, 0)
    plsc.subcore_barrier()
    pltpu.sync_copy(acc_sh.at[pl.ds(base, RPT)],
                    out_hbm.at[c, pl.ds(base, RPT)])


def _tc_mm_body(x_ref, w1_ref, h_ref):
    h_ref[...] = jnp.dot(x_ref[...], w1_ref[...],
                         preferred_element_type=jnp.float32, precision=_P)


def _tc_scale_body(h_ref, degp_ref, hp_ref, dinv_ref):
    deg = degp_ref[0, 0:N, 0:1] + degp_ref[1, 0:N, 0:1] + 1.0
    dinv = lax.rsqrt(jnp.maximum(deg, 1e-12))
    hp_ref[...] = h_ref[...] * dinv
    dinv_ref[...] = dinv


def _tc_b_body(s1_ref, h1_ref, dinv_ref, b1_ref, g_ref, be_ref, w2_ref,
               h2_ref):
    dinv = dinv_ref[...]
    t = (s1_ref[0, 0:N] + s1_ref[1, 0:N] + h1_ref[...]) * dinv + b1_ref[...]
    mu = jnp.mean(t, axis=0, keepdims=True)
    var = jnp.mean((t - mu) * (t - mu), axis=0, keepdims=True)
    y = (t - mu) * lax.rsqrt(var + 1e-5) * g_ref[...] + be_ref[...]
    h2 = jnp.dot(y, w2_ref[...],
                 preferred_element_type=jnp.float32, precision=_P)
    h2_ref[...] = h2 * dinv


def _tc_c_body(s2_ref, h2_ref, dinv_ref, b2_ref, out_ref):
    out_ref[...] = ((s2_ref[0, 0:N] + s2_ref[1, 0:N] + h2_ref[...])
                    * dinv_ref[...] + b2_ref[...])


_tc_mm = pl.pallas_call(
    _tc_mm_body,
    out_shape=jax.ShapeDtypeStruct((N, D), jnp.float32),
)

_tc_scale = pl.pallas_call(
    _tc_scale_body,
    out_shape=[jax.ShapeDtypeStruct((N, D), jnp.float32),
               jax.ShapeDtypeStruct((N, 1), jnp.float32)],
)

_tc_b = pl.pallas_call(
    _tc_b_body,
    out_shape=jax.ShapeDtypeStruct((N, D), jnp.float32),
)

_tc_c = pl.pallas_call(
    _tc_c_body,
    out_shape=jax.ShapeDtypeStruct((N, D), jnp.float32),
)

_SEG_CFG = (80, None, 'serial')
_seg_sc = _make_seg(*_SEG_CFG)


def kernel(quantized_f_embedding, edge_index, W1, b1, gamma, beta, W2, b2):
    x = quantized_f_embedding
    src3, dst3 = _pack_edges(edge_index, *_SEG_CFG)
    b1r = b1.reshape(1, D)
    b2r = b2.reshape(1, D)
    gr = gamma.reshape(1, D)
    ber = beta.reshape(1, D)

    degp = _deg_sc(dst3)
    h1 = _tc_mm(x, W1)
    h1p, dinv = _tc_scale(h1, degp)
    s1p = _seg_sc(h1p, src3, dst3)
    h2p = _tc_b(s1p, h1p, dinv, b1r, gr, ber, W2)
    s2p = _seg_sc(h2p, src3, dst3)
    out = _tc_c(s2p, h2p, dinv, b2r)

    return out
